# Initial kernel scaffold; baseline (speedup 1.0000x reference)
import jax
import jax.numpy as jnp
from jax.experimental import pallas as pl


def kernel(x, edge_index, batch, W1, b1, W2, b2, gate_W, gate_b, bn_gamma, bn_beta, lin2_W, lin2_b):
    raise NotImplementedError("write your pallas kernel here")


# speedup vs baseline: 11.9207x; 11.9207x over previous
import functools

import jax
import jax.numpy as jnp
from jax import lax
from jax.experimental import pallas as pl
from jax.experimental.pallas import tpu as pltpu
from jax.experimental.pallas import tpu_sc as plsc

_NC = 2
_NS = 16
_NW = _NC * _NS

_G = 128


def _sc_aggregate(x, src, dst3d, zeros, *, k):
    n, f = x.shape
    e = src.shape[0]
    epw = e // _NW
    iters = epw // k
    rpt = (n // (_NS * 8)) * 8
    rem = n - _NS * rpt

    mesh = plsc.VectorSubcoreMesh(core_axis_name="c", subcore_axis_name="s")

    @functools.partial(
        pl.kernel,
        mesh=mesh,
        out_type=jax.ShapeDtypeStruct((_NC, n, f), jnp.float32),
        scratch_types=[
            pltpu.VMEM_SHARED((n, f), jnp.float32),
            pltpu.VMEM((epw,), jnp.int32),
            pltpu.VMEM((iters, k), jnp.int32),
            pltpu.VMEM((k, f), jnp.float32),
            pltpu.VMEM((k, f), jnp.float32),
            pltpu.SemaphoreType.DMA,
            pltpu.SemaphoreType.DMA,
        ],
    )
    def body(x_hbm, src_hbm, dst_hbm, zeros_hbm, out_hbm,
             acc, src_v, dst_v, rows0, rows1, sem0, sem1):
        c = lax.axis_index("c")
        s = lax.axis_index("s")
        w = s * _NC + c
        row0 = s * rpt

        @pl.when(c == 0)
        def _():
            pltpu.sync_copy(x_hbm.at[pl.ds(row0, rpt)], acc.at[pl.ds(row0, rpt)])

            @pl.when(s == _NS - 1)
            def _():
                pltpu.sync_copy(x_hbm.at[pl.ds(_NS * rpt, rem)],
                                acc.at[pl.ds(_NS * rpt, rem)])

        @pl.when(c != 0)
        def _():
            pltpu.sync_copy(zeros_hbm.at[pl.ds(row0, rpt)],
                            acc.at[pl.ds(row0, rpt)])

            @pl.when(s == _NS - 1)
            def _():
                pltpu.sync_copy(zeros_hbm.at[pl.ds(_NS * rpt, rem)],
                                acc.at[pl.ds(_NS * rpt, rem)])

        pltpu.sync_copy(src_hbm.at[pl.ds(w * epw, epw)], src_v)
        pltpu.sync_copy(dst_hbm.at[w], dst_v)
        plsc.subcore_barrier()

        pltpu.async_copy(x_hbm.at[src_v.at[pl.ds(0, k)]], rows0, sem0)

        def step(i, carry):
            even = (i % 2) == 0

            @pl.when(i + 1 < iters)
            def _():
                @pl.when(even)
                def _():
                    pltpu.async_copy(
                        x_hbm.at[src_v.at[pl.ds((i + 1) * k, k)]], rows1, sem1)

                @pl.when(jnp.logical_not(even))
                def _():
                    pltpu.async_copy(
                        x_hbm.at[src_v.at[pl.ds((i + 1) * k, k)]], rows0, sem0)

            @pl.when(even)
            def _():
                pltpu.make_async_copy(
                    x_hbm.at[src_v.at[pl.ds(i * k, k)]], rows0, sem0).wait()
                pltpu.sync_copy(rows0, acc.at[dst_v.at[i]], add=True)

            @pl.when(jnp.logical_not(even))
            def _():
                pltpu.make_async_copy(
                    x_hbm.at[src_v.at[pl.ds(i * k, k)]], rows1, sem1).wait()
                pltpu.sync_copy(rows1, acc.at[dst_v.at[i]], add=True)

            return carry

        lax.fori_loop(0, iters, step, 0)
        plsc.subcore_barrier()

        pltpu.sync_copy(acc.at[pl.ds(row0, rpt)],
                        out_hbm.at[c, pl.ds(row0, rpt)])

        @pl.when(s == _NS - 1)
        def _():
            pltpu.sync_copy(acc.at[pl.ds(_NS * rpt, rem)],
                            out_hbm.at[c, pl.ds(_NS * rpt, rem)])

    return body(x, src, dst3d, zeros)


def _tc_body(p_ref, batch_ref, w1_ref, b1_ref, w2_ref, b2_ref, gw_ref,
             gb_ref, gam_ref, bet_ref, lw_ref, lb_ref, o_ref):
    hp = p_ref[0] + p_ref[1]
    h = jnp.dot(hp, w1_ref[...], preferred_element_type=jnp.float32)
    h = jnp.maximum(h + b1_ref[...], 0.0)
    h = jnp.dot(h, w2_ref[...], preferred_element_type=jnp.float32)
    h = jnp.maximum(h + b2_ref[...], 0.0)

    gate = jnp.sum(h * gw_ref[...], axis=1, keepdims=True) + gb_ref[0, 0]

    n = h.shape[0]
    gids = lax.broadcasted_iota(jnp.int32, (1, _G), 1)
    onehot = batch_ref[...] == gids

    neg_inf = jnp.float32(-jnp.inf)
    gmax = jnp.max(jnp.where(onehot, gate, neg_inf), axis=0, keepdims=True)
    gmax = jnp.where(gmax == neg_inf, 0.0, gmax)
    gmax_pn = jnp.sum(jnp.where(onehot, gmax, 0.0), axis=1, keepdims=True)
    e = jnp.exp(gate - gmax_pn)
    denom = jnp.sum(jnp.where(onehot, e, 0.0), axis=0, keepdims=True)
    denom_pn = jnp.sum(jnp.where(onehot, denom, 0.0), axis=1, keepdims=True)
    alpha = e / (denom_pn + 1e-16)

    pooled = lax.dot_general(onehot.astype(jnp.float32), alpha * h,
                             (((0,), (0,)), ((), ())),
                             preferred_element_type=jnp.float32)

    mean = jnp.mean(pooled, axis=0, keepdims=True)
    var = jnp.mean((pooled - mean) ** 2, axis=0, keepdims=True)
    pooled = ((pooled - mean) / jnp.sqrt(var + 1e-5)) * gam_ref[...] \
        + bet_ref[...]

    out = jnp.dot(pooled, lw_ref[...], preferred_element_type=jnp.float32)
    out = out + lb_ref[...]
    m = jnp.max(out, axis=1, keepdims=True)
    z = out - m
    o_ref[...] = z - jnp.log(jnp.sum(jnp.exp(z), axis=1, keepdims=True))


def _tc_tail(parts, batch2d, W1, b1, W2, b2, gate_W, gate_b,
             bn_gamma, bn_beta, lin2_W, lin2_b):
    c = lin2_W.shape[1]
    return pl.pallas_call(
        _tc_body,
        out_shape=jax.ShapeDtypeStruct((_G, c), jnp.float32),
    )(parts, batch2d,
      W1, b1.reshape(1, -1), W2, b2.reshape(1, -1),
      gate_W.reshape(1, -1), gate_b.reshape(1, 1),
      bn_gamma.reshape(1, -1), bn_beta.reshape(1, -1),
      lin2_W, lin2_b.reshape(1, -1))


def kernel(x, edge_index, batch, W1, b1, W2, b2, gate_W, gate_b,
           bn_gamma, bn_beta, lin2_W, lin2_b):
    k = 80
    src = edge_index[0]
    dst3d = edge_index[1].reshape(_NW, -1, k)
    zeros = jnp.zeros_like(x)
    parts = _sc_aggregate(x, src, dst3d, zeros, k=k)
    return _tc_tail(parts, batch.reshape(-1, 1), W1, b1, W2, b2,
                    gate_W, gate_b, bn_gamma, bn_beta, lin2_W, lin2_b)


# speedup vs baseline: 12.6455x; 1.0608x over previous
import functools

import jax
import jax.numpy as jnp
from jax import lax
from jax.experimental import pallas as pl
from jax.experimental.pallas import tpu as pltpu
from jax.experimental.pallas import tpu_sc as plsc

_NC = 2
_NS = 16
_NW = _NC * _NS

_G = 128
_B = 5
_AHEAD = 3


def _sc_aggregate(x, src, dst3d, *, k):
    n, f = x.shape
    e = src.shape[0]
    epw = e // _NW
    nh = 2
    eph = epw // nh
    iters = eph // k
    rpt = (n // (_NS * 8)) * 8
    rem = n - _NS * rpt

    mesh = plsc.VectorSubcoreMesh(core_axis_name="c", subcore_axis_name="s")

    @functools.partial(
        pl.kernel,
        mesh=mesh,
        out_type=jax.ShapeDtypeStruct((_NC, n, f), jnp.float32),
        scratch_types=[
            pltpu.VMEM_SHARED((n, f), jnp.float32),
            pltpu.VMEM((eph,), jnp.int32),
            pltpu.VMEM((iters, k), jnp.int32),
        ] + [pltpu.VMEM((k, f), jnp.float32) for _ in range(_B)]
          + [pltpu.SemaphoreType.DMA for _ in range(2 * _B + 1)],
    )
    def body(x_hbm, src_hbm, dst_hbm, out_hbm, acc, src_v, dst_v, *bufs):
        rows = bufs[:_B]
        sg = bufs[_B:2 * _B]
        ss = bufs[2 * _B:3 * _B]
        sem_i = bufs[3 * _B]

        c = lax.axis_index("c")
        s = lax.axis_index("s")
        w = s * _NC + c
        row0 = s * rpt

        def stage_idx(h):
            pltpu.async_copy(src_hbm.at[pl.ds(w * epw + h * eph, eph)],
                             src_v, sem_i)
            pltpu.async_copy(dst_hbm.at[w * nh + h], dst_v, sem_i)
            pltpu.make_async_copy(src_hbm.at[pl.ds(w * epw + h * eph, eph)],
                                  src_v, sem_i).wait()
            pltpu.make_async_copy(dst_hbm.at[w * nh + h], dst_v, sem_i).wait()

        stage_idx(0)

        pltpu.async_copy(x_hbm.at[pl.ds(row0, rpt)],
                         acc.at[pl.ds(row0, rpt)], sem_i)

        @pl.when(s == _NS - 1)
        def _():
            pltpu.async_copy(x_hbm.at[pl.ds(_NS * rpt, rem)],
                             acc.at[pl.ds(_NS * rpt, rem)], sem_i)

        for u in range(_AHEAD):
            pltpu.async_copy(x_hbm.at[src_v.at[pl.ds(u * k, k)]],
                             rows[u], sg[u])

        pltpu.make_async_copy(x_hbm.at[pl.ds(row0, rpt)],
                              acc.at[pl.ds(row0, rpt)], sem_i).wait()

        @pl.when(s == _NS - 1)
        def _():
            pltpu.make_async_copy(x_hbm.at[pl.ds(_NS * rpt, rem)],
                                  acc.at[pl.ds(_NS * rpt, rem)], sem_i).wait()

        plsc.subcore_barrier()

        def group(g, carry):
            for u in range(_B):
                i = g * _B + u
                pltpu.make_async_copy(x_hbm.at[src_v.at[pl.ds(i * k, k)]],
                                      rows[u], sg[u]).wait()
                pltpu.async_copy(rows[u], acc.at[dst_v.at[i]], ss[u],
                                 add=True)
                v = (u + _AHEAD) % _B
                j = i + _AHEAD

                @pl.when(j < iters)
                def _():
                    @pl.when(j >= _B)
                    def _():
                        pltpu.make_async_copy(
                            rows[v], acc.at[dst_v.at[j - _B]], ss[v]).wait()
                    pltpu.async_copy(x_hbm.at[src_v.at[pl.ds(j * k, k)]],
                                     rows[v], sg[v])
            return carry

        def drain_scatters():
            for u in range(_B):
                pltpu.make_async_copy(rows[u],
                                      acc.at[dst_v.at[iters - _B + u]],
                                      ss[u]).wait()

        for h in range(nh):
            if h > 0:
                stage_idx(h)
                for u in range(_AHEAD):
                    pltpu.async_copy(x_hbm.at[src_v.at[pl.ds(u * k, k)]],
                                     rows[u], sg[u])
            lax.fori_loop(0, iters // _B, group, 0)
            drain_scatters()

        plsc.subcore_barrier()

        pltpu.sync_copy(acc.at[pl.ds(row0, rpt)],
                        out_hbm.at[c, pl.ds(row0, rpt)])

        @pl.when(s == _NS - 1)
        def _():
            pltpu.sync_copy(acc.at[pl.ds(_NS * rpt, rem)],
                            out_hbm.at[c, pl.ds(_NS * rpt, rem)])

    return body(x, src, dst3d)


def _tc_body(p_ref, x_ref, batch_ref, w1_ref, b1_ref, w2_ref, b2_ref, gw_ref,
             gb_ref, gam_ref, bet_ref, lw_ref, lb_ref, o_ref):
    hp = p_ref[0] + p_ref[1] - x_ref[...]
    h = jnp.dot(hp, w1_ref[...], preferred_element_type=jnp.float32)
    h = jnp.maximum(h + b1_ref[...], 0.0)
    h = jnp.dot(h, w2_ref[...], preferred_element_type=jnp.float32)
    h = jnp.maximum(h + b2_ref[...], 0.0)

    gate = jnp.sum(h * gw_ref[...], axis=1, keepdims=True) + gb_ref[0, 0]

    gids = lax.broadcasted_iota(jnp.int32, (1, _G), 1)
    onehot = batch_ref[...] == gids

    neg_inf = jnp.float32(-jnp.inf)
    gmax = jnp.max(jnp.where(onehot, gate, neg_inf), axis=0, keepdims=True)
    gmax = jnp.where(gmax == neg_inf, 0.0, gmax)
    gmax_pn = jnp.sum(jnp.where(onehot, gmax, 0.0), axis=1, keepdims=True)
    e = jnp.exp(gate - gmax_pn)
    denom = jnp.sum(jnp.where(onehot, e, 0.0), axis=0, keepdims=True)
    denom_pn = jnp.sum(jnp.where(onehot, denom, 0.0), axis=1, keepdims=True)
    alpha = e / (denom_pn + 1e-16)

    pooled = lax.dot_general(onehot.astype(jnp.float32), alpha * h,
                             (((0,), (0,)), ((), ())),
                             preferred_element_type=jnp.float32)

    mean = jnp.mean(pooled, axis=0, keepdims=True)
    var = jnp.mean((pooled - mean) ** 2, axis=0, keepdims=True)
    pooled = ((pooled - mean) / jnp.sqrt(var + 1e-5)) * gam_ref[...] \
        + bet_ref[...]

    out = jnp.dot(pooled, lw_ref[...], preferred_element_type=jnp.float32)
    out = out + lb_ref[...]
    m = jnp.max(out, axis=1, keepdims=True)
    z = out - m
    o_ref[...] = z - jnp.log(jnp.sum(jnp.exp(z), axis=1, keepdims=True))


def _tc_tail(parts, x, batch2d, W1, b1, W2, b2, gate_W, gate_b,
             bn_gamma, bn_beta, lin2_W, lin2_b):
    c = lin2_W.shape[1]
    return pl.pallas_call(
        _tc_body,
        out_shape=jax.ShapeDtypeStruct((_G, c), jnp.float32),
    )(parts, x, batch2d,
      W1, b1.reshape(1, -1), W2, b2.reshape(1, -1),
      gate_W.reshape(1, -1), gate_b.reshape(1, 1),
      bn_gamma.reshape(1, -1), bn_beta.reshape(1, -1),
      lin2_W, lin2_b.reshape(1, -1))


def kernel(x, edge_index, batch, W1, b1, W2, b2, gate_W, gate_b,
           bn_gamma, bn_beta, lin2_W, lin2_b):
    k = 40
    src = edge_index[0]
    dst3d = edge_index[1].reshape(_NW * 2, -1, k)
    parts = _sc_aggregate(x, src, dst3d, k=k)
    return _tc_tail(parts, x, batch.reshape(-1, 1), W1, b1, W2, b2,
                    gate_W, gate_b, bn_gamma, bn_beta, lin2_W, lin2_b)


# speedup vs baseline: 13.4584x; 1.0643x over previous
import functools

import jax
import jax.numpy as jnp
from jax import lax
from jax.experimental import pallas as pl
from jax.experimental.pallas import tpu as pltpu
from jax.experimental.pallas import tpu_sc as plsc

_NC = 2
_NS = 16
_NW = _NC * _NS

_G = 128
_B = 5
_AHEAD = 4


def _sc_aggregate(x, src, dst3d, *, k):
    n, f = x.shape
    e = src.shape[0]
    epw = e // _NW
    nh = 2
    eph = epw // nh
    iters = eph // k
    rpt = (n // (_NS * 8)) * 8
    rem = n - _NS * rpt

    mesh = plsc.VectorSubcoreMesh(core_axis_name="c", subcore_axis_name="s")

    @functools.partial(
        pl.kernel,
        mesh=mesh,
        out_type=jax.ShapeDtypeStruct((_NC, n, f), jnp.float32),
        scratch_types=[
            pltpu.VMEM_SHARED((n, f), jnp.float32),
            pltpu.VMEM((eph,), jnp.int32),
            pltpu.VMEM((iters, k), jnp.int32),
        ] + [pltpu.VMEM((k, f), jnp.float32) for _ in range(_B)]
          + [pltpu.SemaphoreType.DMA for _ in range(2 * _B + 1)],
    )
    def body(x_hbm, src_hbm, dst_hbm, out_hbm, acc, src_v, dst_v, *bufs):
        rows = bufs[:_B]
        sg = bufs[_B:2 * _B]
        ss = bufs[2 * _B:3 * _B]
        sem_i = bufs[3 * _B]

        c = lax.axis_index("c")
        s = lax.axis_index("s")
        w = s * _NC + c
        row0 = s * rpt

        def stage_idx(h):
            pltpu.async_copy(src_hbm.at[pl.ds(w * epw + h * eph, eph)],
                             src_v, sem_i)
            pltpu.async_copy(dst_hbm.at[w * nh + h], dst_v, sem_i)
            pltpu.make_async_copy(src_hbm.at[pl.ds(w * epw + h * eph, eph)],
                                  src_v, sem_i).wait()
            pltpu.make_async_copy(dst_hbm.at[w * nh + h], dst_v, sem_i).wait()

        stage_idx(0)

        pltpu.async_copy(x_hbm.at[pl.ds(row0, rpt)],
                         acc.at[pl.ds(row0, rpt)], sem_i)

        @pl.when(s == _NS - 1)
        def _():
            pltpu.async_copy(x_hbm.at[pl.ds(_NS * rpt, rem)],
                             acc.at[pl.ds(_NS * rpt, rem)], sem_i)

        for u in range(_AHEAD):
            pltpu.async_copy(x_hbm.at[src_v.at[pl.ds(u * k, k)]],
                             rows[u], sg[u])

        pltpu.make_async_copy(x_hbm.at[pl.ds(row0, rpt)],
                              acc.at[pl.ds(row0, rpt)], sem_i).wait()

        @pl.when(s == _NS - 1)
        def _():
            pltpu.make_async_copy(x_hbm.at[pl.ds(_NS * rpt, rem)],
                                  acc.at[pl.ds(_NS * rpt, rem)], sem_i).wait()

        plsc.subcore_barrier()

        def group(g, carry):
            for u in range(_B):
                i = g * _B + u

                @pl.when(i < iters)
                def _():
                    pltpu.make_async_copy(
                        x_hbm.at[src_v.at[pl.ds(i * k, k)]],
                        rows[u], sg[u]).wait()
                    pltpu.async_copy(rows[u], acc.at[dst_v.at[i]], ss[u],
                                     add=True)
                    v = (u + _AHEAD) % _B
                    j = i + _AHEAD

                    @pl.when(j < iters)
                    def _():
                        @pl.when(j >= _B)
                        def _():
                            pltpu.make_async_copy(
                                rows[v], acc.at[dst_v.at[j - _B]],
                                ss[v]).wait()
                        pltpu.async_copy(
                            x_hbm.at[src_v.at[pl.ds(j * k, k)]],
                            rows[v], sg[v])
            return carry

        def drain_scatters():
            for u in range(_B):
                pltpu.make_async_copy(rows[u],
                                      acc.at[dst_v.at[iters - _B + u]],
                                      ss[u]).wait()

        for h in range(nh):
            if h > 0:
                stage_idx(h)
                for u in range(_AHEAD):
                    pltpu.async_copy(x_hbm.at[src_v.at[pl.ds(u * k, k)]],
                                     rows[u], sg[u])
            lax.fori_loop(0, (iters + _B - 1) // _B, group, 0)
            drain_scatters()

        plsc.subcore_barrier()

        pltpu.sync_copy(acc.at[pl.ds(row0, rpt)],
                        out_hbm.at[c, pl.ds(row0, rpt)])

        @pl.when(s == _NS - 1)
        def _():
            pltpu.sync_copy(acc.at[pl.ds(_NS * rpt, rem)],
                            out_hbm.at[c, pl.ds(_NS * rpt, rem)])

    return body(x, src, dst3d)


def _tc_body(p_ref, x_ref, batch_ref, w1_ref, b1_ref, w2_ref, b2_ref, gw_ref,
             gb_ref, gam_ref, bet_ref, lw_ref, lb_ref, o_ref):
    hp = p_ref[0] + p_ref[1] - x_ref[...]
    h = jnp.dot(hp, w1_ref[...], preferred_element_type=jnp.float32)
    h = jnp.maximum(h + b1_ref[...], 0.0)
    h = jnp.dot(h, w2_ref[...], preferred_element_type=jnp.float32)
    h = jnp.maximum(h + b2_ref[...], 0.0)

    gate = jnp.sum(h * gw_ref[...], axis=1, keepdims=True) + gb_ref[0, 0]

    gids = lax.broadcasted_iota(jnp.int32, (1, _G), 1)
    onehot = batch_ref[...] == gids

    neg_inf = jnp.float32(-jnp.inf)
    gmax = jnp.max(jnp.where(onehot, gate, neg_inf), axis=0, keepdims=True)
    gmax = jnp.where(gmax == neg_inf, 0.0, gmax)
    gmax_pn = jnp.sum(jnp.where(onehot, gmax, 0.0), axis=1, keepdims=True)
    e = jnp.exp(gate - gmax_pn)
    denom = jnp.sum(jnp.where(onehot, e, 0.0), axis=0, keepdims=True)
    denom_pn = jnp.sum(jnp.where(onehot, denom, 0.0), axis=1, keepdims=True)
    alpha = e / (denom_pn + 1e-16)

    pooled = lax.dot_general(onehot.astype(jnp.float32), alpha * h,
                             (((0,), (0,)), ((), ())),
                             preferred_element_type=jnp.float32)

    mean = jnp.mean(pooled, axis=0, keepdims=True)
    var = jnp.mean((pooled - mean) ** 2, axis=0, keepdims=True)
    pooled = ((pooled - mean) / jnp.sqrt(var + 1e-5)) * gam_ref[...] \
        + bet_ref[...]

    out = jnp.dot(pooled, lw_ref[...], preferred_element_type=jnp.float32)
    out = out + lb_ref[...]
    m = jnp.max(out, axis=1, keepdims=True)
    z = out - m
    o_ref[...] = z - jnp.log(jnp.sum(jnp.exp(z), axis=1, keepdims=True))


def _tc_tail(parts, x, batch2d, W1, b1, W2, b2, gate_W, gate_b,
             bn_gamma, bn_beta, lin2_W, lin2_b):
    c = lin2_W.shape[1]
    return pl.pallas_call(
        _tc_body,
        out_shape=jax.ShapeDtypeStruct((_G, c), jnp.float32),
    )(parts, x, batch2d,
      W1, b1.reshape(1, -1), W2, b2.reshape(1, -1),
      gate_W.reshape(1, -1), gate_b.reshape(1, 1),
      bn_gamma.reshape(1, -1), bn_beta.reshape(1, -1),
      lin2_W, lin2_b.reshape(1, -1))


def kernel(x, edge_index, batch, W1, b1, W2, b2, gate_W, gate_b,
           bn_gamma, bn_beta, lin2_W, lin2_b):
    k = 40
    src = edge_index[0]
    dst3d = edge_index[1].reshape(_NW * 2, -1, k)
    parts = _sc_aggregate(x, src, dst3d, k=k)
    return _tc_tail(parts, x, batch.reshape(-1, 1), W1, b1, W2, b2,
                    gate_W, gate_b, bn_gamma, bn_beta, lin2_W, lin2_b)


# speedup vs baseline: 14.0058x; 1.0407x over previous
import functools

import jax
import jax.numpy as jnp
from jax import lax
from jax.experimental import pallas as pl
from jax.experimental.pallas import tpu as pltpu
from jax.experimental.pallas import tpu_sc as plsc

_NC = 2
_NS = 16
_NW = _NC * _NS

_G = 128
_B = 5
_AHEAD = 4


def _sc_aggregate(x, src, dst1d, *, k):
    n, f = x.shape
    e = src.shape[0]
    epw = e // _NW
    nh = 2
    eph = epw // nh
    iters = eph // k
    rpt = (n // (_NS * 8)) * 8
    rem = n - _NS * rpt

    mesh = plsc.VectorSubcoreMesh(core_axis_name="c", subcore_axis_name="s")

    @functools.partial(
        pl.kernel,
        mesh=mesh,
        out_type=jax.ShapeDtypeStruct((_NC, n, f), jnp.float32),
        scratch_types=[
            pltpu.VMEM_SHARED((n, f), jnp.float32),
            pltpu.VMEM((eph,), jnp.int32),
            pltpu.VMEM((eph,), jnp.int32),
        ] + [pltpu.VMEM((k, f), jnp.float32) for _ in range(_B)]
          + [pltpu.SemaphoreType.DMA for _ in range(2 * _B + 1)],
    )
    def body(x_hbm, src_hbm, dst_hbm, out_hbm, acc, src_v, dst_v, *bufs):
        rows = bufs[:_B]
        sg = bufs[_B:2 * _B]
        ss = bufs[2 * _B:3 * _B]
        sem_i = bufs[3 * _B]

        c = lax.axis_index("c")
        s = lax.axis_index("s")
        w = s * _NC + c
        row0 = s * rpt

        def stage_idx(h):
            pltpu.async_copy(src_hbm.at[pl.ds(w * epw + h * eph, eph)],
                             src_v, sem_i)
            pltpu.async_copy(dst_hbm.at[pl.ds(w * epw + h * eph, eph)],
                             dst_v, sem_i)
            pltpu.make_async_copy(src_hbm.at[pl.ds(w * epw + h * eph, eph)],
                                  src_v, sem_i).wait()
            pltpu.make_async_copy(dst_hbm.at[pl.ds(w * epw + h * eph, eph)],
                                  dst_v, sem_i).wait()

        stage_idx(0)

        pltpu.async_copy(x_hbm.at[pl.ds(row0, rpt)],
                         acc.at[pl.ds(row0, rpt)], sem_i)

        @pl.when(s == _NS - 1)
        def _():
            pltpu.async_copy(x_hbm.at[pl.ds(_NS * rpt, rem)],
                             acc.at[pl.ds(_NS * rpt, rem)], sem_i)

        for u in range(_AHEAD):
            pltpu.async_copy(x_hbm.at[src_v.at[pl.ds(u * k, k)]],
                             rows[u], sg[u])

        pltpu.make_async_copy(x_hbm.at[pl.ds(row0, rpt)],
                              acc.at[pl.ds(row0, rpt)], sem_i).wait()

        @pl.when(s == _NS - 1)
        def _():
            pltpu.make_async_copy(x_hbm.at[pl.ds(_NS * rpt, rem)],
                                  acc.at[pl.ds(_NS * rpt, rem)], sem_i).wait()

        plsc.subcore_barrier()

        def group(g, carry):
            for u in range(_B):
                i = g * _B + u

                @pl.when(i < iters)
                def _():
                    pltpu.make_async_copy(
                        x_hbm.at[src_v.at[pl.ds(i * k, k)]],
                        rows[u], sg[u]).wait()
                    pltpu.async_copy(rows[u], acc.at[dst_v.at[pl.ds(i * k, k)]], ss[u],
                                     add=True)
                    v = (u + _AHEAD) % _B
                    j = i + _AHEAD

                    @pl.when(j < iters)
                    def _():
                        @pl.when(j >= _B)
                        def _():
                            pltpu.make_async_copy(
                                rows[v], acc.at[dst_v.at[pl.ds((j - _B) * k, k)]],
                                ss[v]).wait()
                        pltpu.async_copy(
                            x_hbm.at[src_v.at[pl.ds(j * k, k)]],
                            rows[v], sg[v])
            return carry

        def drain_scatters():
            for u in range(_B):
                pltpu.make_async_copy(rows[u],
                                      acc.at[dst_v.at[pl.ds((iters - _B + u) * k, k)]],
                                      ss[u]).wait()

        for h in range(nh):
            if h > 0:
                stage_idx(h)
                for u in range(_AHEAD):
                    pltpu.async_copy(x_hbm.at[src_v.at[pl.ds(u * k, k)]],
                                     rows[u], sg[u])
            lax.fori_loop(0, (iters + _B - 1) // _B, group, 0)
            drain_scatters()

        plsc.subcore_barrier()

        pltpu.sync_copy(acc.at[pl.ds(row0, rpt)],
                        out_hbm.at[c, pl.ds(row0, rpt)])

        @pl.when(s == _NS - 1)
        def _():
            pltpu.sync_copy(acc.at[pl.ds(_NS * rpt, rem)],
                            out_hbm.at[c, pl.ds(_NS * rpt, rem)])

    return body(x, src, dst1d)


def _tc_body(p_ref, x_ref, batch_ref, w1_ref, b1_ref, w2_ref, b2_ref, gw_ref,
             gb_ref, gam_ref, bet_ref, lw_ref, lb_ref, o_ref):
    hp = p_ref[0] + p_ref[1] - x_ref[...]
    h = jnp.dot(hp, w1_ref[...], preferred_element_type=jnp.float32)
    h = jnp.maximum(h + b1_ref[...], 0.0)
    h = jnp.dot(h, w2_ref[...], preferred_element_type=jnp.float32)
    h = jnp.maximum(h + b2_ref[...], 0.0)

    gate = jnp.sum(h * gw_ref[...], axis=1, keepdims=True) + gb_ref[0, 0]

    gids = lax.broadcasted_iota(jnp.int32, (1, _G), 1)
    onehot = batch_ref[...] == gids

    neg_inf = jnp.float32(-jnp.inf)
    gmax = jnp.max(jnp.where(onehot, gate, neg_inf), axis=0, keepdims=True)
    gmax = jnp.where(gmax == neg_inf, 0.0, gmax)
    gmax_pn = jnp.sum(jnp.where(onehot, gmax, 0.0), axis=1, keepdims=True)
    e = jnp.exp(gate - gmax_pn)
    denom = jnp.sum(jnp.where(onehot, e, 0.0), axis=0, keepdims=True)
    denom_pn = jnp.sum(jnp.where(onehot, denom, 0.0), axis=1, keepdims=True)
    alpha = e / (denom_pn + 1e-16)

    pooled = lax.dot_general(onehot.astype(jnp.float32), alpha * h,
                             (((0,), (0,)), ((), ())),
                             preferred_element_type=jnp.float32)

    mean = jnp.mean(pooled, axis=0, keepdims=True)
    var = jnp.mean((pooled - mean) ** 2, axis=0, keepdims=True)
    pooled = ((pooled - mean) / jnp.sqrt(var + 1e-5)) * gam_ref[...] \
        + bet_ref[...]

    out = jnp.dot(pooled, lw_ref[...], preferred_element_type=jnp.float32)
    out = out + lb_ref[...]
    m = jnp.max(out, axis=1, keepdims=True)
    z = out - m
    o_ref[...] = z - jnp.log(jnp.sum(jnp.exp(z), axis=1, keepdims=True))


def _tc_tail(parts, x, batch2d, W1, b1, W2, b2, gate_W, gate_b,
             bn_gamma, bn_beta, lin2_W, lin2_b):
    c = lin2_W.shape[1]
    return pl.pallas_call(
        _tc_body,
        out_shape=jax.ShapeDtypeStruct((_G, c), jnp.float32),
    )(parts, x, batch2d,
      W1, b1.reshape(1, -1), W2, b2.reshape(1, -1),
      gate_W.reshape(1, -1), gate_b.reshape(1, 1),
      bn_gamma.reshape(1, -1), bn_beta.reshape(1, -1),
      lin2_W, lin2_b.reshape(1, -1))


def kernel(x, edge_index, batch, W1, b1, W2, b2, gate_W, gate_b,
           bn_gamma, bn_beta, lin2_W, lin2_b):
    k = 40
    src = edge_index[0]
    dst1d = edge_index[1]
    parts = _sc_aggregate(x, src, dst1d, k=k)
    return _tc_tail(parts, x, batch.reshape(-1, 1), W1, b1, W2, b2,
                    gate_W, gate_b, bn_gamma, bn_beta, lin2_W, lin2_b)
